# Initial kernel scaffold; baseline (speedup 1.0000x reference)
import functools

import jax
import jax.numpy as jnp
from jax import lax
from jax.experimental import pallas as pl
from jax.experimental.pallas import tpu as pltpu
from jax.experimental.pallas import tpu_sc as plsc

_NUM_EMB = 1000000
_DIM = 32
_BATCH = 4096
_SEQ = 200
_B = _BATCH * _SEQ

_info = plsc.get_sparse_core_info()
_NC, _NS = _info.num_cores, _info.num_subcores
_NW = _NC * _NS
_BPW = _B // _NW
_CH = 128
_NCH = _BPW // _CH

_mesh = plsc.VectorSubcoreMesh(core_axis_name="c", subcore_axis_name="s")


@functools.partial(
    pl.kernel,
    mesh=_mesh,
    out_type=jax.ShapeDtypeStruct((_B, _DIM), jnp.float32),
    scratch_types=[
        pltpu.VMEM((_NCH, _CH), jnp.int32),
        pltpu.VMEM((_CH, _DIM), jnp.float32),
        pltpu.SemaphoreType.DMA,
    ],
)
def _gather_sc(table_hbm, idx_hbm, out_hbm, idx_v, rows_v, sem):
    wid = lax.axis_index("s") * _NC + lax.axis_index("c")
    base = wid * _BPW
    pltpu.sync_copy(idx_hbm.at[wid], idx_v)

    def body(j, carry):
        pltpu.async_copy(table_hbm.at[idx_v.at[j]], rows_v, sem).wait()
        pltpu.sync_copy(rows_v, out_hbm.at[pl.ds(base + j * _CH, _CH)])
        return carry

    lax.fori_loop(0, _NCH, body, 0)


def kernel(input_ids, weight):
    idx = input_ids.reshape(_NW, _NCH, _CH)
    out = _gather_sc(weight, idx)
    return out.reshape(_BATCH, _SEQ, _DIM)


# speedup vs baseline: 1.3078x; 1.3078x over previous
import functools

import jax
import jax.numpy as jnp
from jax import lax
from jax.experimental import pallas as pl
from jax.experimental.pallas import tpu as pltpu
from jax.experimental.pallas import tpu_sc as plsc

_NUM_EMB = 1000000
_DIM = 32
_BATCH = 4096
_SEQ = 200
_B = _BATCH * _SEQ

_info = plsc.get_sparse_core_info()
_NC, _NS = _info.num_cores, _info.num_subcores
_NW = _NC * _NS
_BPW = _B // _NW
_CH = 128
_NCH = _BPW // _CH

_mesh = plsc.VectorSubcoreMesh(core_axis_name="c", subcore_axis_name="s")


@functools.partial(
    pl.kernel,
    mesh=_mesh,
    out_type=jax.ShapeDtypeStruct((_B, _DIM), jnp.float32),
    scratch_types=[
        pltpu.VMEM((_NCH, _CH), jnp.int32),
        pltpu.VMEM((_CH, _DIM), jnp.float32),
        pltpu.SemaphoreType.DMA,
    ],
    compiler_params=pltpu.CompilerParams(use_tc_tiling_on_sc=False),
)
def _gather_sc(table_hbm, idx_hbm, out_hbm, idx_v, rows_v, sem):
    wid = lax.axis_index("s") * _NC + lax.axis_index("c")
    base = wid * _BPW
    pltpu.sync_copy(idx_hbm.at[wid], idx_v)

    def body(j, carry):
        pltpu.async_copy(table_hbm.at[idx_v.at[j]], rows_v, sem).wait()
        pltpu.sync_copy(rows_v, out_hbm.at[pl.ds(base + j * _CH, _CH)])
        return carry

    lax.fori_loop(0, _NCH, body, 0)


def kernel(input_ids, weight):
    idx = input_ids.reshape(_NW, _NCH, _CH)
    out = _gather_sc(weight, idx)
    return out.reshape(_BATCH, _SEQ, _DIM)


# speedup vs baseline: 1.4941x; 1.1425x over previous
import functools

import jax
import jax.numpy as jnp
from jax import lax
from jax.experimental import pallas as pl
from jax.experimental.pallas import tpu as pltpu
from jax.experimental.pallas import tpu_sc as plsc

_NUM_EMB = 1000000
_DIM = 32
_BATCH = 4096
_SEQ = 200
_B = _BATCH * _SEQ

_info = plsc.get_sparse_core_info()
_NC, _NS = _info.num_cores, _info.num_subcores
_NW = _NC * _NS
_BPW = _B // _NW
_CH = 128
_NCH = _BPW // _CH
_K = 10
_SB = _K * _CH
_SK = _NCH // _K

_mesh = plsc.VectorSubcoreMesh(core_axis_name="c", subcore_axis_name="s")


@functools.partial(
    pl.kernel,
    mesh=_mesh,
    out_type=jax.ShapeDtypeStruct((_B, _DIM), jnp.float32),
    scratch_types=[
        pltpu.VMEM((_NCH, _CH), jnp.int32),
        pltpu.VMEM((_SB, _DIM), jnp.float32),
        pltpu.VMEM((_SB, _DIM), jnp.float32),
        pltpu.SemaphoreType.DMA,
        pltpu.SemaphoreType.DMA,
        pltpu.SemaphoreType.DMA,
        pltpu.SemaphoreType.DMA,
    ],
    compiler_params=pltpu.CompilerParams(use_tc_tiling_on_sc=False),
)
def _gather_sc(table_hbm, idx_hbm, out_hbm, idx_v, rows0, rows1,
               semg0, semg1, sems0, sems1):
    wid = lax.axis_index("s") * _NC + lax.axis_index("c")
    base = wid * _BPW
    rows = (rows0, rows1)
    semg = (semg0, semg1)
    sems = (sems0, sems1)

    pltpu.sync_copy(idx_hbm.at[wid], idx_v)

    def fire_g(t, buf):
        for b in range(_K):
            pltpu.async_copy(
                table_hbm.at[idx_v.at[t * _K + b]],
                rows[buf].at[pl.ds(b * _CH, _CH)],
                semg[buf],
            )

    def drain_g(buf):
        pltpu.make_async_copy(
            table_hbm.at[pl.ds(0, _SB)], rows[buf], semg[buf]
        ).wait()

    def fire_s(t, buf):
        pltpu.async_copy(
            rows[buf], out_hbm.at[pl.ds(base + t * _SB, _SB)], sems[buf]
        )

    def drain_s(buf):
        pltpu.make_async_copy(
            rows[buf], out_hbm.at[pl.ds(0, _SB)], sems[buf]
        ).wait()

    fire_g(0, 0)
    drain_g(0)
    fire_s(0, 0)
    fire_g(1, 1)

    @pl.loop(1, _SK - 1, step=2)
    def _steady(t0):
        for d in range(2):
            t = t0 + d
            buf = (1 + d) % 2
            nbuf = 1 - buf
            drain_g(buf)
            fire_s(t, buf)
            drain_s(nbuf)
            fire_g(t + 1, nbuf)

    drain_g(1)
    fire_s(_SK - 1, 1)
    drain_s(0)
    drain_s(1)


def kernel(input_ids, weight):
    idx = input_ids.reshape(_NW, _NCH, _CH)
    out = _gather_sc(weight, idx)
    return out.reshape(_BATCH, _SEQ, _DIM)


# speedup vs baseline: 1.4955x; 1.0009x over previous
import functools

import jax
import jax.numpy as jnp
from jax import lax
from jax.experimental import pallas as pl
from jax.experimental.pallas import tpu as pltpu
from jax.experimental.pallas import tpu_sc as plsc

_NUM_EMB = 1000000
_DIM = 32
_BATCH = 4096
_SEQ = 200
_B = _BATCH * _SEQ

_info = plsc.get_sparse_core_info()
_NC, _NS = _info.num_cores, _info.num_subcores
_NW = _NC * _NS
_BPW = _B // _NW
_CH = 1280
_NCH = _BPW // _CH
_K = 1
_SB = _K * _CH
_SK = _NCH // _K

_mesh = plsc.VectorSubcoreMesh(core_axis_name="c", subcore_axis_name="s")


@functools.partial(
    pl.kernel,
    mesh=_mesh,
    out_type=jax.ShapeDtypeStruct((_B, _DIM), jnp.float32),
    scratch_types=[
        pltpu.VMEM((_NCH, _CH), jnp.int32),
        pltpu.VMEM((_SB, _DIM), jnp.float32),
        pltpu.VMEM((_SB, _DIM), jnp.float32),
        pltpu.SemaphoreType.DMA,
        pltpu.SemaphoreType.DMA,
        pltpu.SemaphoreType.DMA,
        pltpu.SemaphoreType.DMA,
    ],
    compiler_params=pltpu.CompilerParams(use_tc_tiling_on_sc=False),
)
def _gather_sc(table_hbm, idx_hbm, out_hbm, idx_v, rows0, rows1,
               semg0, semg1, sems0, sems1):
    wid = lax.axis_index("s") * _NC + lax.axis_index("c")
    base = wid * _BPW
    rows = (rows0, rows1)
    semg = (semg0, semg1)
    sems = (sems0, sems1)

    pltpu.sync_copy(idx_hbm.at[wid], idx_v)

    def fire_g(t, buf):
        for b in range(_K):
            pltpu.async_copy(
                table_hbm.at[idx_v.at[t * _K + b]],
                rows[buf].at[pl.ds(b * _CH, _CH)],
                semg[buf],
            )

    def drain_g(buf):
        pltpu.make_async_copy(
            table_hbm.at[pl.ds(0, _SB)], rows[buf], semg[buf]
        ).wait()

    def fire_s(t, buf):
        pltpu.async_copy(
            rows[buf], out_hbm.at[pl.ds(base + t * _SB, _SB)], sems[buf]
        )

    def drain_s(buf):
        pltpu.make_async_copy(
            rows[buf], out_hbm.at[pl.ds(0, _SB)], sems[buf]
        ).wait()

    fire_g(0, 0)
    drain_g(0)
    fire_s(0, 0)
    fire_g(1, 1)

    @pl.loop(1, _SK - 1, step=2)
    def _steady(t0):
        for d in range(2):
            t = t0 + d
            buf = (1 + d) % 2
            nbuf = 1 - buf
            drain_g(buf)
            fire_s(t, buf)
            drain_s(nbuf)
            fire_g(t + 1, nbuf)

    drain_g(1)
    fire_s(_SK - 1, 1)
    drain_s(0)
    drain_s(1)


def kernel(input_ids, weight):
    idx = input_ids.reshape(_NW, _NCH, _CH)
    out = _gather_sc(weight, idx)
    return out.reshape(_BATCH, _SEQ, _DIM)
